# baseline (speedup 1.0000x reference)
import functools

import numpy as np

import jax
import jax.numpy as jnp
from jax import lax
from jax.experimental import pallas as pl
from jax.experimental.pallas import tpu as pltpu
from jax.experimental.pallas import tpu_sc as plsc

N = 10000
E = 320000
C = 128

NC = 2
NS = 16
NW = NC * NS
B = 64
K = 160
EPT = K * B
E_PAD = NW * EPT
N_PAD = 10240
RPT = N_PAD // NS

_PERM = np.zeros((C,), dtype=np.int32)
for _k in range(C // 32):
    for _i in range(16):
        _PERM[32 * _k + 2 * _i] = 32 * _k + _i
        _PERM[32 * _k + 2 * _i + 1] = 32 * _k + 16 + _i

_MESH = plsc.VectorSubcoreMesh(
    core_axis_name="c", subcore_axis_name="s", num_cores=NC, num_subcores=NS)


@functools.partial(
    pl.kernel,
    mesh=_MESH,
    out_type=jax.ShapeDtypeStruct((NC, N_PAD, 16), jnp.float32),
    scratch_types=[
        pltpu.VMEM((B, 16), jnp.float32),
        [pltpu.VMEM((B,), jnp.int32) for _ in range(4)],
        pltpu.VMEM((16, 16), jnp.float32),
        [pltpu.SemaphoreType.DMA for _ in range(4)],
        [pltpu.SemaphoreType.DMA for _ in range(2)],
        pltpu.VMEM_SHARED((N_PAD, 16), jnp.float32),
    ],
)
def _deg_kernel(dst_hbm, out_hbm, ones_v, didxs, zbuf, si, ssem, acc_sh):
    c = lax.axis_index("c")
    s = lax.axis_index("s")
    wid = s * NC + c

    def fill_ones(i, carry):
        ones_v[i, :] = jnp.ones((16,), jnp.float32)
        return carry

    lax.fori_loop(0, B, fill_ones, 0)

    def fill_zero(i, carry):
        zbuf[i, :] = jnp.zeros((16,), jnp.float32)
        return carry

    lax.fori_loop(0, 16, fill_zero, 0)

    def zero_acc(k, carry):
        pltpu.sync_copy(zbuf, acc_sh.at[pl.ds(s * RPT + k * 16, 16)])
        return carry

    lax.fori_loop(0, RPT // 16, zero_acc, 0)
    plsc.subcore_barrier()

    def idx_load(j, p):
        pltpu.async_copy(dst_hbm.at[wid, j], didxs[p], si[p])

    def idx_wait(j, p):
        pltpu.make_async_copy(dst_hbm.at[wid, j], didxs[p], si[p]).wait()

    def scat(p):
        pltpu.async_copy(ones_v, acc_sh.at[didxs[p]], ssem[p % 2], add=True)

    def scat_wait(p):
        pltpu.make_async_copy(ones_v, acc_sh.at[didxs[p]], ssem[p % 2]).wait()

    idx_load(0, 0)
    idx_load(1, 1)

    def body(t, carry):
        for q in range(4):
            j = 4 * t + q
            if q < 2:
                @pl.when(t > 0)
                def _():
                    scat_wait((q + 2) % 4)
            else:
                scat_wait((q + 2) % 4)
            if q < 2:
                idx_load(j + 2, (q + 2) % 4)
            else:
                @pl.when(t < K // 4 - 1)
                def _():
                    idx_load(j + 2, (q + 2) % 4)
            idx_wait(j, q)
            scat(q)
        return carry

    lax.fori_loop(0, K // 4, body, 0)
    scat_wait(2)
    scat_wait(3)
    plsc.subcore_barrier()
    pltpu.sync_copy(acc_sh.at[pl.ds(s * RPT, RPT)],
                    out_hbm.at[c, pl.ds(s * RPT, RPT)])


@functools.partial(
    pl.kernel,
    mesh=_MESH,
    compiler_params=pltpu.CompilerParams(use_tc_tiling_on_sc=False),
    out_type=jax.ShapeDtypeStruct((NC, N_PAD, C), jnp.float32),
    scratch_types=[
        [pltpu.VMEM((B,), jnp.int32) for _ in range(4)],
        [pltpu.VMEM((B,), jnp.int32) for _ in range(4)],
        [pltpu.VMEM((B, C // 2), jnp.int32) for _ in range(2)],
        [pltpu.VMEM((B, C), jnp.float32) for _ in range(2)],
        [pltpu.SemaphoreType.DMA for _ in range(4)],
        [pltpu.SemaphoreType.DMA for _ in range(2)],
        [pltpu.SemaphoreType.DMA for _ in range(2)],
        pltpu.VMEM_SHARED((N_PAD, C), jnp.float32),
    ],
)
def _msg_kernel(src_hbm, dst_hbm, ybf_hbm, out_hbm,
                sidxs, didxs, rbf, rf32, si, sg, ss, acc_sh):
    c = lax.axis_index("c")
    s = lax.axis_index("s")
    base = (s * NC + c) * K

    def fill_zero(i, carry):
        for jj in range(C // 16):
            rf32[0][i, pl.ds(jj * 16, 16)] = jnp.zeros((16,), jnp.float32)
        return carry

    lax.fori_loop(0, B, fill_zero, 0)

    def zero_acc(k, carry):
        pltpu.sync_copy(rf32[0], acc_sh.at[pl.ds(s * RPT + k * B, B)])
        return carry

    lax.fori_loop(0, RPT // B, zero_acc, 0)
    plsc.subcore_barrier()

    def idx_load(j, p):
        pltpu.async_copy(src_hbm.at[base + j], sidxs[p], si[p])
        pltpu.async_copy(dst_hbm.at[base + j], didxs[p], si[p])

    def idx_wait(j, p):
        pltpu.make_async_copy(src_hbm.at[base + j], sidxs[p], si[p]).wait()
        pltpu.make_async_copy(dst_hbm.at[base + j], didxs[p], si[p]).wait()

    def gather(p, r):
        pltpu.async_copy(ybf_hbm.at[sidxs[p]], rbf[r], sg[r])

    def gather_wait(p, r):
        pltpu.make_async_copy(ybf_hbm.at[sidxs[p]], rbf[r], sg[r]).wait()

    sh16 = jnp.full((16,), 16, jnp.int32)
    msk = jnp.full((16,), -65536, jnp.int32)

    def convert(r):
        def row4(m, carry):
            i0 = 4 * m
            for di in range(4):
                i = i0 + di
                for kk in range(C // 32):
                    v = rbf[r][i, pl.ds(kk * 16, 16)]
                    lo = lax.bitcast_convert_type(lax.shift_left(v, sh16),
                                                  jnp.float32)
                    hi = lax.bitcast_convert_type(lax.bitwise_and(v, msk),
                                                  jnp.float32)
                    rf32[r][i, pl.ds(kk * 32, 16)] = lo
                    rf32[r][i, pl.ds(kk * 32 + 16, 16)] = hi
            return carry

        lax.fori_loop(0, B // 4, row4, 0)

    def scat(p, r):
        pltpu.async_copy(rf32[r], acc_sh.at[didxs[p]], ss[r], add=True)

    def scat_wait(p, r):
        pltpu.make_async_copy(rf32[r], acc_sh.at[didxs[p]], ss[r]).wait()

    idx_load(0, 0)
    idx_load(1, 1)

    def body(t, carry):
        for q in range(4):
            j = 4 * t + q
            if q < 2:
                @pl.when(t > 0)
                def _():
                    scat_wait((q + 2) % 4, q % 2)
            else:
                scat_wait((q + 2) % 4, q % 2)
            if q < 2:
                idx_load(j + 2, (q + 2) % 4)
            else:
                @pl.when(t < K // 4 - 1)
                def _():
                    idx_load(j + 2, (q + 2) % 4)
            idx_wait(j, q)
            gather(q, q % 2)
            if q == 0:
                @pl.when(t > 0)
                def _():
                    gather_wait(3, 1)
                    convert(1)
                    scat(3, 1)
            else:
                gather_wait(q - 1, (q - 1) % 2)
                convert((q - 1) % 2)
                scat(q - 1, (q - 1) % 2)
        return carry

    lax.fori_loop(0, K // 4, body, 0)
    gather_wait(3, 1)
    convert(1)
    scat(3, 1)
    scat_wait(2, 0)
    scat_wait(3, 1)
    plsc.subcore_barrier()
    pltpu.sync_copy(acc_sh.at[pl.ds(s * RPT, RPT)],
                    out_hbm.at[c, pl.ds(s * RPT, RPT)])


def _mm_body(f_ref, w_ref, d0_ref, d1_ref, y_ref, dinv_ref):
    d = d0_ref[...] + d1_ref[...] + 1.0
    dinv = lax.rsqrt(d)
    x = jnp.dot(f_ref[...], w_ref[...], preferred_element_type=jnp.float32)
    y_ref[...] = x * dinv
    dinv_ref[...] = dinv


def _fin_body(s_ref, y_ref, dinv_ref, b_ref, o_ref):
    t = (s_ref[0] + s_ref[1] + y_ref[...]) * dinv_ref[...] + b_ref[...]
    o_ref[...] = jnp.maximum(t, 0.0)


_RB = 400


def kernel(feats, edges, W, b):
    src = edges[0].astype(jnp.int32)
    dst = edges[1].astype(jnp.int32)
    pad = E_PAD - E
    srcp = jnp.concatenate([src, jnp.zeros((pad,), jnp.int32)])
    dstp = jnp.concatenate([dst, jnp.full((pad,), N, jnp.int32)])
    dst3 = dstp.reshape(NW, K, B)
    src2 = srcp.reshape(E_PAD // B, B)
    dst2 = dstp.reshape(E_PAD // B, B)

    degp = _deg_kernel(dst3)
    d0 = degp[0, :N, 0:1]
    d1 = degp[1, :N, 0:1]

    y, dinv = pl.pallas_call(
        _mm_body,
        grid=(N // _RB,),
        in_specs=[
            pl.BlockSpec((_RB, C), lambda i: (i, 0)),
            pl.BlockSpec((C, C), lambda i: (0, 0)),
            pl.BlockSpec((_RB, 1), lambda i: (i, 0)),
            pl.BlockSpec((_RB, 1), lambda i: (i, 0)),
        ],
        out_specs=[
            pl.BlockSpec((_RB, C), lambda i: (i, 0)),
            pl.BlockSpec((_RB, 1), lambda i: (i, 0)),
        ],
        out_shape=[
            jax.ShapeDtypeStruct((N, C), jnp.float32),
            jax.ShapeDtypeStruct((N, 1), jnp.float32),
        ],
    )(feats, W, d0, d1)

    ybf = y.astype(jnp.bfloat16)[:, _PERM]
    ybi = lax.bitcast_convert_type(ybf.reshape(N, C // 2, 2), jnp.int32)

    s_parts = _msg_kernel(src2, dst2, ybi)

    out = pl.pallas_call(
        _fin_body,
        grid=(N // _RB,),
        in_specs=[
            pl.BlockSpec((2, _RB, C), lambda i: (0, i, 0)),
            pl.BlockSpec((_RB, C), lambda i: (i, 0)),
            pl.BlockSpec((_RB, 1), lambda i: (i, 0)),
            pl.BlockSpec((1, C), lambda i: (0, 0)),
        ],
        out_specs=pl.BlockSpec((_RB, C), lambda i: (i, 0)),
        out_shape=jax.ShapeDtypeStruct((N, C), jnp.float32),
    )(s_parts, y, dinv, b.reshape(1, C))
    return out


# speedup vs baseline: 1.0909x; 1.0909x over previous
import functools

import numpy as np

import jax
import jax.numpy as jnp
from jax import lax
from jax.experimental import pallas as pl
from jax.experimental.pallas import tpu as pltpu
from jax.experimental.pallas import tpu_sc as plsc

N = 10000
E = 320000
C = 128

NC = 2
NS = 16
NW = NC * NS
B = 64
K = 160
EPT = K * B
E_PAD = NW * EPT
N_PAD = 10240
RPT = N_PAD // NS

_PERM = np.zeros((C,), dtype=np.int32)
for _k in range(C // 32):
    for _i in range(16):
        _PERM[32 * _k + 2 * _i] = 32 * _k + _i
        _PERM[32 * _k + 2 * _i + 1] = 32 * _k + 16 + _i

_MESH = plsc.VectorSubcoreMesh(
    core_axis_name="c", subcore_axis_name="s", num_cores=NC, num_subcores=NS)


@functools.partial(
    pl.kernel,
    mesh=_MESH,
    out_type=jax.ShapeDtypeStruct((NC, N_PAD, 16), jnp.float32),
    scratch_types=[
        pltpu.VMEM((B, 16), jnp.float32),
        [pltpu.VMEM((B,), jnp.int32) for _ in range(4)],
        pltpu.VMEM((16, 16), jnp.float32),
        [pltpu.SemaphoreType.DMA for _ in range(4)],
        [pltpu.SemaphoreType.DMA for _ in range(2)],
        pltpu.VMEM_SHARED((N_PAD, 16), jnp.float32),
    ],
)
def _deg_kernel(dst_hbm, out_hbm, ones_v, didxs, zbuf, si, ssem, acc_sh):
    c = lax.axis_index("c")
    s = lax.axis_index("s")
    wid = s * NC + c

    def fill_ones(i, carry):
        ones_v[i, :] = jnp.ones((16,), jnp.float32)
        return carry

    lax.fori_loop(0, B, fill_ones, 0)

    def fill_zero(i, carry):
        zbuf[i, :] = jnp.zeros((16,), jnp.float32)
        return carry

    lax.fori_loop(0, 16, fill_zero, 0)

    def zero_acc(k, carry):
        pltpu.sync_copy(zbuf, acc_sh.at[pl.ds(s * RPT + k * 16, 16)])
        return carry

    lax.fori_loop(0, RPT // 16, zero_acc, 0)
    plsc.subcore_barrier()

    def idx_load(j, p):
        pltpu.async_copy(dst_hbm.at[wid, j], didxs[p], si[p])

    def idx_wait(j, p):
        pltpu.make_async_copy(dst_hbm.at[wid, j], didxs[p], si[p]).wait()

    def scat(p):
        pltpu.async_copy(ones_v, acc_sh.at[didxs[p]], ssem[p % 2], add=True)

    def scat_wait(p):
        pltpu.make_async_copy(ones_v, acc_sh.at[didxs[p]], ssem[p % 2]).wait()

    idx_load(0, 0)
    idx_load(1, 1)

    def body(t, carry):
        for q in range(4):
            j = 4 * t + q
            if q < 2:
                @pl.when(t > 0)
                def _():
                    scat_wait((q + 2) % 4)
            else:
                scat_wait((q + 2) % 4)
            if q < 2:
                idx_load(j + 2, (q + 2) % 4)
            else:
                @pl.when(t < K // 4 - 1)
                def _():
                    idx_load(j + 2, (q + 2) % 4)
            idx_wait(j, q)
            scat(q)
        return carry

    lax.fori_loop(0, K // 4, body, 0)
    scat_wait(2)
    scat_wait(3)
    plsc.subcore_barrier()
    pltpu.sync_copy(acc_sh.at[pl.ds(s * RPT, RPT)],
                    out_hbm.at[c, pl.ds(s * RPT, RPT)])


@functools.partial(
    pl.kernel,
    mesh=_MESH,
    compiler_params=pltpu.CompilerParams(use_tc_tiling_on_sc=False),
    out_type=jax.ShapeDtypeStruct((NC, N_PAD, C), jnp.float32),
    scratch_types=[
        [pltpu.VMEM((B,), jnp.int32) for _ in range(4)],
        [pltpu.VMEM((B,), jnp.int32) for _ in range(4)],
        [pltpu.VMEM((B, C // 2), jnp.int32) for _ in range(2)],
        [pltpu.VMEM((B, C), jnp.float32) for _ in range(2)],
        [pltpu.SemaphoreType.DMA for _ in range(4)],
        [pltpu.SemaphoreType.DMA for _ in range(2)],
        [pltpu.SemaphoreType.DMA for _ in range(2)],
        pltpu.VMEM_SHARED((N_PAD, C), jnp.float32),
    ],
)
def _msg_kernel(src_hbm, dst_hbm, ybf_hbm, out_hbm,
                sidxs, didxs, rbf, rf32, si, sg, ss, acc_sh):
    c = lax.axis_index("c")
    s = lax.axis_index("s")
    base = (s * NC + c) * K

    def fill_zero(i, carry):
        for jj in range(C // 16):
            rf32[0][i, pl.ds(jj * 16, 16)] = jnp.zeros((16,), jnp.float32)
        return carry

    lax.fori_loop(0, B, fill_zero, 0)

    def zero_acc(k, carry):
        pltpu.sync_copy(rf32[0], acc_sh.at[pl.ds(s * RPT + k * B, B)])
        return carry

    lax.fori_loop(0, RPT // B, zero_acc, 0)
    plsc.subcore_barrier()

    def idx_load(j, p):
        pltpu.async_copy(src_hbm.at[base + j], sidxs[p], si[p])
        pltpu.async_copy(dst_hbm.at[base + j], didxs[p], si[p])

    def idx_wait(j, p):
        pltpu.make_async_copy(src_hbm.at[base + j], sidxs[p], si[p]).wait()
        pltpu.make_async_copy(dst_hbm.at[base + j], didxs[p], si[p]).wait()

    def gather(p, r):
        pltpu.async_copy(ybf_hbm.at[sidxs[p]], rbf[r], sg[r])

    def gather_wait(p, r):
        pltpu.make_async_copy(ybf_hbm.at[sidxs[p]], rbf[r], sg[r]).wait()

    def convert(r):
        sh16 = jnp.full((16,), 16, jnp.int32)
        msk = jnp.full((16,), -65536, jnp.int32)

        for i in range(B):
            for kk in range(C // 32):
                v = rbf[r][i, pl.ds(kk * 16, 16)]
                lo = lax.bitcast_convert_type(lax.shift_left(v, sh16),
                                              jnp.float32)
                hi = lax.bitcast_convert_type(lax.bitwise_and(v, msk),
                                              jnp.float32)
                rf32[r][i, pl.ds(kk * 32, 16)] = lo
                rf32[r][i, pl.ds(kk * 32 + 16, 16)] = hi

    def scat(p, r):
        pltpu.async_copy(rf32[r], acc_sh.at[didxs[p]], ss[r], add=True)

    def scat_wait(p, r):
        pltpu.make_async_copy(rf32[r], acc_sh.at[didxs[p]], ss[r]).wait()

    idx_load(0, 0)
    idx_load(1, 1)

    def body(t, carry):
        for q in range(4):
            j = 4 * t + q
            if q < 2:
                @pl.when(t > 0)
                def _():
                    scat_wait((q + 2) % 4, q % 2)
            else:
                scat_wait((q + 2) % 4, q % 2)
            if q < 2:
                idx_load(j + 2, (q + 2) % 4)
            else:
                @pl.when(t < K // 4 - 1)
                def _():
                    idx_load(j + 2, (q + 2) % 4)
            idx_wait(j, q)
            gather(q, q % 2)
            if q == 0:
                @pl.when(t > 0)
                def _():
                    gather_wait(3, 1)
                    convert(1)
                    scat(3, 1)
            else:
                gather_wait(q - 1, (q - 1) % 2)
                convert((q - 1) % 2)
                scat(q - 1, (q - 1) % 2)
        return carry

    lax.fori_loop(0, K // 4, body, 0)
    gather_wait(3, 1)
    convert(1)
    scat(3, 1)
    scat_wait(2, 0)
    scat_wait(3, 1)
    plsc.subcore_barrier()
    pltpu.sync_copy(acc_sh.at[pl.ds(s * RPT, RPT)],
                    out_hbm.at[c, pl.ds(s * RPT, RPT)])


def _mm_body(f_ref, w_ref, d0_ref, d1_ref, y_ref, dinv_ref):
    d = d0_ref[...] + d1_ref[...] + 1.0
    dinv = lax.rsqrt(d)
    x = jnp.dot(f_ref[...], w_ref[...], preferred_element_type=jnp.float32)
    y_ref[...] = x * dinv
    dinv_ref[...] = dinv


def _fin_body(s_ref, y_ref, dinv_ref, b_ref, o_ref):
    t = (s_ref[0] + s_ref[1] + y_ref[...]) * dinv_ref[...] + b_ref[...]
    o_ref[...] = jnp.maximum(t, 0.0)


_RB = 400


def kernel(feats, edges, W, b):
    src = edges[0].astype(jnp.int32)
    dst = edges[1].astype(jnp.int32)
    pad = E_PAD - E
    srcp = jnp.concatenate([src, jnp.zeros((pad,), jnp.int32)])
    dstp = jnp.concatenate([dst, jnp.full((pad,), N, jnp.int32)])
    dst3 = dstp.reshape(NW, K, B)
    src2 = srcp.reshape(E_PAD // B, B)
    dst2 = dstp.reshape(E_PAD // B, B)

    degp = _deg_kernel(dst3)
    d0 = degp[0, :N, 0:1]
    d1 = degp[1, :N, 0:1]

    y, dinv = pl.pallas_call(
        _mm_body,
        grid=(N // _RB,),
        in_specs=[
            pl.BlockSpec((_RB, C), lambda i: (i, 0)),
            pl.BlockSpec((C, C), lambda i: (0, 0)),
            pl.BlockSpec((_RB, 1), lambda i: (i, 0)),
            pl.BlockSpec((_RB, 1), lambda i: (i, 0)),
        ],
        out_specs=[
            pl.BlockSpec((_RB, C), lambda i: (i, 0)),
            pl.BlockSpec((_RB, 1), lambda i: (i, 0)),
        ],
        out_shape=[
            jax.ShapeDtypeStruct((N, C), jnp.float32),
            jax.ShapeDtypeStruct((N, 1), jnp.float32),
        ],
    )(feats, W, d0, d1)

    ybf = y.astype(jnp.bfloat16)[:, _PERM]
    ybi = lax.bitcast_convert_type(ybf.reshape(N, C // 2, 2), jnp.int32)

    s_parts = _msg_kernel(src2, dst2, ybi)

    out = pl.pallas_call(
        _fin_body,
        grid=(N // _RB,),
        in_specs=[
            pl.BlockSpec((2, _RB, C), lambda i: (0, i, 0)),
            pl.BlockSpec((_RB, C), lambda i: (i, 0)),
            pl.BlockSpec((_RB, 1), lambda i: (i, 0)),
            pl.BlockSpec((1, C), lambda i: (0, 0)),
        ],
        out_specs=pl.BlockSpec((_RB, C), lambda i: (i, 0)),
        out_shape=jax.ShapeDtypeStruct((N, C), jnp.float32),
    )(s_parts, y, dinv, b.reshape(1, C))
    return out
